# baseline (speedup 1.0000x reference)
import jax
import jax.numpy as jnp
from jax import lax
from jax.experimental import pallas as pl
from jax.experimental.pallas import tpu as pltpu
from jax.experimental.pallas import tpu_sc as plsc

_H, _W = 128, 128
_B, _M, _N = 16, 512, 128
_HW = _H * _W
_NHALF = _N // 2
_L = 16


def _sc_scatter(idx_hbm, xt_hbm, out_hbm, cell_v, xt_v, buf_a, buf_b,
                sem_a, sem_b):
    nc = 2
    wid = lax.axis_index("s") * nc + lax.axis_index("c")
    b = wid // 2
    nlo = (wid % 2) * _NHALF

    pltpu.sync_copy(idx_hbm.at[b], cell_v)
    pltpu.sync_copy(xt_hbm.at[b, pl.ds(nlo, _NHALF)], xt_v)

    def zero_body(i, _):
        buf_a[pl.ds(i * _L, _L)] = jnp.zeros((_L,), jnp.float32)
        buf_b[pl.ds(i * _L, _L)] = jnp.zeros((_L,), jnp.float32)
        return 0

    lax.fori_loop(0, _HW // _L, zero_body, 0)

    def scatter_row(row, buf):
        def m_body(mi, _):
            idx = cell_v[pl.ds(mi * _L, _L)]
            val = xt_v[row, pl.ds(mi * _L, _L)]
            plsc.store_scatter(buf, [idx], val)
            return 0

        lax.fori_loop(0, _M // _L, m_body, 0)

    def row_pair(r2, _):
        row0 = 2 * r2
        row1 = 2 * r2 + 1

        @pl.when(r2 > 0)
        def _():
            pltpu.make_async_copy(
                buf_a, out_hbm.at[b, nlo + row0 - 2], sem_a).wait()

        scatter_row(row0, buf_a)
        pltpu.async_copy(buf_a, out_hbm.at[b, nlo + row0], sem_a)

        @pl.when(r2 > 0)
        def _():
            pltpu.make_async_copy(
                buf_b, out_hbm.at[b, nlo + row1 - 2], sem_b).wait()

        scatter_row(row1, buf_b)
        pltpu.async_copy(buf_b, out_hbm.at[b, nlo + row1], sem_b)
        return 0

    lax.fori_loop(0, _NHALF // 2, row_pair, 0)

    pltpu.make_async_copy(
        buf_a, out_hbm.at[b, nlo + _NHALF - 2], sem_a).wait()
    pltpu.make_async_copy(
        buf_b, out_hbm.at[b, nlo + _NHALF - 1], sem_b).wait()


def kernel(x, spatial_size, location):
    B, M, N = x.shape
    H, W = _H, _W
    HW = H * W
    index = (location[:, :, 0] * spatial_size[1] + location[:, :, 1]) % HW
    index = index.astype(jnp.int32)
    xt = jnp.transpose(x, (0, 2, 1))

    mesh = plsc.VectorSubcoreMesh(core_axis_name="c", subcore_axis_name="s")
    scatter = pl.kernel(
        _sc_scatter,
        mesh=mesh,
        out_type=jax.ShapeDtypeStruct((B, N, HW), jnp.float32),
        scratch_types=[
            pltpu.VMEM((M,), jnp.int32),
            pltpu.VMEM((_NHALF, M), jnp.float32),
            pltpu.VMEM((HW,), jnp.float32),
            pltpu.VMEM((HW,), jnp.float32),
            pltpu.SemaphoreType.DMA,
            pltpu.SemaphoreType.DMA,
        ],
        compiler_params=pltpu.CompilerParams(needs_layout_passes=False),
    )
    out = scatter(index, xt)
    return out.reshape(B, N, H, W)
